# baseline (speedup 1.0000x reference)
import functools
import jax
import jax.numpy as jnp
from jax import lax
from jax.experimental import pallas as pl
from jax.experimental.pallas import tpu as pltpu, tpu_sc as plsc

_ROWS = 8192
_COLS = 128

_INFO = plsc.get_sparse_core_info()
_NC, _NS = _INFO.num_cores, _INFO.num_subcores
_NW = _NC * _NS
_RPW = _ROWS // _NW


def _sc_copy(table_hbm, out_hbm, rows_v, sem):
    wid = lax.axis_index("s") * _NC + lax.axis_index("c")
    base = wid * _RPW
    pltpu.async_copy(table_hbm.at[pl.ds(base, _RPW)], rows_v, sem).wait()
    pltpu.async_copy(rows_v, out_hbm.at[pl.ds(base, _RPW)], sem).wait()


def kernel(x, pos_table):
    del x
    mesh = plsc.VectorSubcoreMesh(core_axis_name="c", subcore_axis_name="s")
    k = functools.partial(
        pl.kernel,
        mesh=mesh,
        out_type=jax.ShapeDtypeStruct((_ROWS, _COLS), jnp.float32),
        scratch_types=[
            pltpu.VMEM((_RPW, _COLS), jnp.float32),
            pltpu.SemaphoreType.DMA,
        ],
    )(_sc_copy)
    return k(pos_table)


# speedup vs baseline: 5.8998x; 5.8998x over previous
import jax
import jax.numpy as jnp
from jax.experimental import pallas as pl

_ROWS = 8192
_COLS = 128
_BLOCK_ROWS = 4096


def _copy_block(t_ref, o_ref):
    o_ref[...] = t_ref[...]


def kernel(x, pos_table):
    del x
    n_blocks = _ROWS // _BLOCK_ROWS
    return pl.pallas_call(
        _copy_block,
        out_shape=jax.ShapeDtypeStruct((_ROWS, _COLS), pos_table.dtype),
        grid=(n_blocks,),
        in_specs=[pl.BlockSpec((_BLOCK_ROWS, _COLS), lambda i: (i, 0))],
        out_specs=pl.BlockSpec((_BLOCK_ROWS, _COLS), lambda i: (i, 0)),
    )(pos_table)


# speedup vs baseline: 6.2082x; 1.0523x over previous
import jax
import jax.numpy as jnp
from jax.experimental import pallas as pl
from jax.experimental.pallas import tpu as pltpu

_ROWS = 8192
_COLS = 128
_N_CHUNKS = 8
_CHUNK = _ROWS // _N_CHUNKS


def _copy_pipeline(t_hbm, o_hbm, buf, *sems):
    in_sems = sems[:_N_CHUNKS]
    out_sems = sems[_N_CHUNKS:]
    ins = [
        pltpu.make_async_copy(
            t_hbm.at[pl.ds(c * _CHUNK, _CHUNK), :],
            buf.at[pl.ds(c * _CHUNK, _CHUNK), :],
            in_sems[c],
        )
        for c in range(_N_CHUNKS)
    ]
    outs = [
        pltpu.make_async_copy(
            buf.at[pl.ds(c * _CHUNK, _CHUNK), :],
            o_hbm.at[pl.ds(c * _CHUNK, _CHUNK), :],
            out_sems[c],
        )
        for c in range(_N_CHUNKS)
    ]
    for c in range(_N_CHUNKS):
        ins[c].start()
    for c in range(_N_CHUNKS):
        ins[c].wait()
        outs[c].start()
    for c in range(_N_CHUNKS):
        outs[c].wait()


def kernel(x, pos_table):
    del x
    return pl.pallas_call(
        _copy_pipeline,
        out_shape=jax.ShapeDtypeStruct((_ROWS, _COLS), pos_table.dtype),
        in_specs=[pl.BlockSpec(memory_space=pl.ANY)],
        out_specs=pl.BlockSpec(memory_space=pl.ANY),
        scratch_shapes=[pltpu.VMEM((_ROWS, _COLS), jnp.float32)]
        + [pltpu.SemaphoreType.DMA] * (2 * _N_CHUNKS),
    )(pos_table)
